# baseline (speedup 1.0000x reference)
import functools
import math

import jax
import jax.numpy as jnp
from jax import lax
from jax.experimental import pallas as pl
from jax.experimental.pallas import tpu as pltpu
from jax.experimental.pallas import tpu_sc as plsc

D_EMB = 64
LANES = 16
I_PER = 2
CHUNKS = ((0, 104), (104, 96))
SCALE = math.sqrt(D_EMB)


@functools.partial(jax.jit)
def _emb_lookup(x, table):
    n_i, n_j = x.shape
    info = plsc.get_sparse_core_info()
    nc, ns = info.num_cores, info.num_subcores
    nw = nc * ns
    i_per_w = n_i // nw
    iters_per_w = i_per_w // I_PER

    mesh = plsc.VectorSubcoreMesh(core_axis_name="c", subcore_axis_name="s")

    @functools.partial(
        pl.kernel,
        mesh=mesh,
        out_type=jax.ShapeDtypeStruct((n_i, n_j, D_EMB), jnp.float32),
        scratch_types=[
            pltpu.VMEM((I_PER, n_j), jnp.int32),
            pltpu.VMEM((I_PER, n_j, D_EMB), jnp.float32),
            pltpu.SemaphoreType.DMA,
        ],
        compiler_params=pltpu.CompilerParams(use_tc_tiling_on_sc=False),
    )
    def k(x_hbm, table_hbm, out_hbm, idx_v, rows_v, sem):
        wid = lax.axis_index("s") * nc + lax.axis_index("c")
        i_base = wid * i_per_w

        def iter_body(it, carry):
            i0 = i_base + it * I_PER
            pltpu.sync_copy(x_hbm.at[pl.ds(i0, I_PER)], idx_v)
            handles = []
            for ii in range(I_PER):
                for off, sz in CHUNKS:
                    handles.append(
                        pltpu.async_copy(
                            table_hbm.at[idx_v.at[ii, pl.ds(off, sz)]],
                            rows_v.at[ii, pl.ds(off, sz)],
                            sem,
                        )
                    )
            for hd in handles:
                hd.wait()

            def scale_body(r, c2):
                for ii in range(I_PER):
                    for c in range(D_EMB // LANES):
                        sl = pl.ds(c * LANES, LANES)
                        rows_v[ii, r, sl] = rows_v[ii, r, sl] * SCALE
                return c2

            lax.fori_loop(0, n_j, scale_body, 0)
            pltpu.sync_copy(rows_v, out_hbm.at[pl.ds(i0, I_PER)])
            return carry

        lax.fori_loop(0, iters_per_w, iter_body, 0)

    return k(x, table)


def kernel(x, table):
    return _emb_lookup(x, table)
